# initial kernel scaffold (unmeasured)
import jax
import jax.numpy as jnp
from jax import lax
from jax.experimental import pallas as pl
from jax.experimental.pallas import tpu as pltpu


def kernel(x, pi):
    def body(x_ref, pi_ref, out_ref, send_sem, recv_sem, copy_sem):
        my_x = lax.axis_index("x")
        my_y = lax.axis_index("y")
        dst_x = pi_ref[my_x]

        @pl.when(dst_x != my_x)
        def _():
            rdma = pltpu.make_async_remote_copy(
                src_ref=x_ref,
                dst_ref=out_ref,
                send_sem=send_sem,
                recv_sem=recv_sem,
                device_id=(dst_x, my_y),
                device_id_type=pl.DeviceIdType.MESH,
            )
            rdma.start()
            rdma.wait()

        @pl.when(dst_x == my_x)
        def _():
            cp = pltpu.make_async_copy(x_ref, out_ref, copy_sem)
            cp.start()
            cp.wait()

    return pl.pallas_call(
        body,
        out_shape=jax.ShapeDtypeStruct(x.shape, x.dtype),
        in_specs=[
            pl.BlockSpec(memory_space=pltpu.ANY),
            pl.BlockSpec(memory_space=pltpu.SMEM),
        ],
        out_specs=pl.BlockSpec(memory_space=pltpu.ANY),
        scratch_shapes=[
            pltpu.SemaphoreType.DMA,
            pltpu.SemaphoreType.DMA,
            pltpu.SemaphoreType.DMA,
        ],
        compiler_params=pltpu.CompilerParams(collective_id=0),
    )(x, pi)


# baseline (device time: 390918 ns/iter reference)
import jax
import jax.numpy as jnp
from jax import lax
from jax.experimental import pallas as pl
from jax.experimental.pallas import tpu as pltpu


def kernel(x, pi):
    def body(x_ref, pi_ref, out_ref, send_sem, recv_sem, copy_sem):
        my_x = lax.axis_index("x")
        my_y = lax.axis_index("y")
        dst_x = pi_ref[my_x]

        @pl.when(dst_x != my_x)
        def _():
            rdma = pltpu.make_async_remote_copy(
                src_ref=x_ref,
                dst_ref=out_ref,
                send_sem=send_sem,
                recv_sem=recv_sem,
                device_id=(dst_x, my_y),
                device_id_type=pl.DeviceIdType.MESH,
            )
            rdma.start()
            rdma.wait()

        @pl.when(dst_x == my_x)
        def _():
            cp = pltpu.make_async_copy(x_ref, out_ref, copy_sem)
            cp.start()
            cp.wait()

    return pl.pallas_call(
        body,
        out_shape=jax.ShapeDtypeStruct(x.shape, x.dtype),
        in_specs=[
            pl.BlockSpec(memory_space=pl.ANY),
            pl.BlockSpec(memory_space=pltpu.MemorySpace.SMEM),
        ],
        out_specs=pl.BlockSpec(memory_space=pl.ANY),
        scratch_shapes=[
            pltpu.SemaphoreType.DMA,
            pltpu.SemaphoreType.DMA,
            pltpu.SemaphoreType.DMA,
        ],
    )(x, pi)


# device time: 225876 ns/iter; 1.7307x vs baseline; 1.7307x over previous
import jax
import jax.numpy as jnp
from jax import lax
from jax.experimental import pallas as pl
from jax.experimental.pallas import tpu as pltpu

C = 16


def kernel(x, pi):
    _, M, N = x.shape
    half = M // 2
    rows = half // C

    def body(x_ref, pi_ref, out_ref, x_send, x_recv, y_send, y_recv, copy_sem):
        my_x = lax.axis_index("x")
        my_y = lax.axis_index("y")
        dst_x = pi_ref[my_x]

        @pl.when(dst_x != my_x)
        def _():
            base = my_y * half
            other = (1 - my_y) * half

            def x_desc(k):
                s = base + k * rows
                return pltpu.make_async_remote_copy(
                    src_ref=x_ref.at[0, pl.ds(s, rows), :],
                    dst_ref=out_ref.at[0, pl.ds(s, rows), :],
                    send_sem=x_send.at[k],
                    recv_sem=x_recv.at[k],
                    device_id=(dst_x, my_y),
                    device_id_type=pl.DeviceIdType.MESH,
                )

            def y_fwd(k):
                s = base + k * rows
                return pltpu.make_async_remote_copy(
                    src_ref=out_ref.at[0, pl.ds(s, rows), :],
                    dst_ref=out_ref.at[0, pl.ds(s, rows), :],
                    send_sem=y_send.at[k],
                    recv_sem=y_recv.at[k],
                    device_id=(my_x, 1 - my_y),
                    device_id_type=pl.DeviceIdType.MESH,
                )

            def y_in(k):
                s = other + k * rows
                return pltpu.make_async_remote_copy(
                    src_ref=out_ref.at[0, pl.ds(s, rows), :],
                    dst_ref=out_ref.at[0, pl.ds(s, rows), :],
                    send_sem=y_send.at[k],
                    recv_sem=y_recv.at[k],
                    device_id=(my_x, 1 - my_y),
                    device_id_type=pl.DeviceIdType.MESH,
                )

            for k in range(C):
                x_desc(k).start()
            for k in range(C):
                x_desc(k).wait_recv()
                y_fwd(k).start()
            for k in range(C):
                y_in(k).wait_recv()
            for k in range(C):
                x_desc(k).wait_send()
                y_fwd(k).wait_send()

        @pl.when(dst_x == my_x)
        def _():
            cp = pltpu.make_async_copy(x_ref, out_ref, copy_sem)
            cp.start()
            cp.wait()

    return pl.pallas_call(
        body,
        out_shape=jax.ShapeDtypeStruct(x.shape, x.dtype),
        in_specs=[
            pl.BlockSpec(memory_space=pl.ANY),
            pl.BlockSpec(memory_space=pltpu.MemorySpace.SMEM),
        ],
        out_specs=pl.BlockSpec(memory_space=pl.ANY),
        scratch_shapes=[
            pltpu.SemaphoreType.DMA((C,)),
            pltpu.SemaphoreType.DMA((C,)),
            pltpu.SemaphoreType.DMA((C,)),
            pltpu.SemaphoreType.DMA((C,)),
            pltpu.SemaphoreType.DMA,
        ],
    )(x, pi)


# device time: 220869 ns/iter; 1.7699x vs baseline; 1.0227x over previous
import jax
import jax.numpy as jnp
from jax import lax
from jax.experimental import pallas as pl
from jax.experimental.pallas import tpu as pltpu

C = 32


def kernel(x, pi):
    _, M, N = x.shape
    half = M // 2
    rows = half // C

    def body(x_ref, pi_ref, out_ref, x_send, x_recv, y_send, y_recv, copy_sem):
        my_x = lax.axis_index("x")
        my_y = lax.axis_index("y")
        dst_x = pi_ref[my_x]

        @pl.when(dst_x != my_x)
        def _():
            base = my_y * half
            other = (1 - my_y) * half

            def x_desc(k):
                s = base + k * rows
                return pltpu.make_async_remote_copy(
                    src_ref=x_ref.at[0, pl.ds(s, rows), :],
                    dst_ref=out_ref.at[0, pl.ds(s, rows), :],
                    send_sem=x_send.at[k],
                    recv_sem=x_recv.at[k],
                    device_id=(dst_x, my_y),
                    device_id_type=pl.DeviceIdType.MESH,
                )

            def y_fwd(k):
                s = base + k * rows
                return pltpu.make_async_remote_copy(
                    src_ref=out_ref.at[0, pl.ds(s, rows), :],
                    dst_ref=out_ref.at[0, pl.ds(s, rows), :],
                    send_sem=y_send.at[k],
                    recv_sem=y_recv.at[k],
                    device_id=(my_x, 1 - my_y),
                    device_id_type=pl.DeviceIdType.MESH,
                )

            def y_in(k):
                s = other + k * rows
                return pltpu.make_async_remote_copy(
                    src_ref=out_ref.at[0, pl.ds(s, rows), :],
                    dst_ref=out_ref.at[0, pl.ds(s, rows), :],
                    send_sem=y_send.at[k],
                    recv_sem=y_recv.at[k],
                    device_id=(my_x, 1 - my_y),
                    device_id_type=pl.DeviceIdType.MESH,
                )

            for k in range(C):
                x_desc(k).start()
            for k in range(C):
                x_desc(k).wait_recv()
                y_fwd(k).start()
            for k in range(C):
                y_in(k).wait_recv()
            for k in range(C):
                x_desc(k).wait_send()
                y_fwd(k).wait_send()

        @pl.when(dst_x == my_x)
        def _():
            cp = pltpu.make_async_copy(x_ref, out_ref, copy_sem)
            cp.start()
            cp.wait()

    return pl.pallas_call(
        body,
        out_shape=jax.ShapeDtypeStruct(x.shape, x.dtype),
        in_specs=[
            pl.BlockSpec(memory_space=pl.ANY),
            pl.BlockSpec(memory_space=pltpu.MemorySpace.SMEM),
        ],
        out_specs=pl.BlockSpec(memory_space=pl.ANY),
        scratch_shapes=[
            pltpu.SemaphoreType.DMA((C,)),
            pltpu.SemaphoreType.DMA((C,)),
            pltpu.SemaphoreType.DMA((C,)),
            pltpu.SemaphoreType.DMA((C,)),
            pltpu.SemaphoreType.DMA,
        ],
    )(x, pi)


# device time: 217395 ns/iter; 1.7982x vs baseline; 1.0160x over previous
import jax
import jax.numpy as jnp
from jax import lax
from jax.experimental import pallas as pl
from jax.experimental.pallas import tpu as pltpu

C = 32


def kernel(x, pi):
    _, M, N = x.shape
    half = M // 2
    rows = half // C

    def body(x_ref, pi_ref, out_ref, x_send, x_recv, y_send, y_recv, copy_sem):
        my_x = lax.axis_index("x")
        my_y = lax.axis_index("y")
        dst_x = pi_ref[my_x]

        barrier = pltpu.get_barrier_semaphore()
        pl.semaphore_signal(barrier, inc=1, device_id=(1 - my_x, my_y),
                            device_id_type=pl.DeviceIdType.MESH)
        pl.semaphore_signal(barrier, inc=1, device_id=(my_x, 1 - my_y),
                            device_id_type=pl.DeviceIdType.MESH)
        pl.semaphore_wait(barrier, 2)

        @pl.when(dst_x != my_x)
        def _():
            base = my_y * half
            other = (1 - my_y) * half

            def x_desc(k):
                s = base + k * rows
                return pltpu.make_async_remote_copy(
                    src_ref=x_ref.at[0, pl.ds(s, rows), :],
                    dst_ref=out_ref.at[0, pl.ds(s, rows), :],
                    send_sem=x_send.at[k],
                    recv_sem=x_recv.at[k],
                    device_id=(dst_x, my_y),
                    device_id_type=pl.DeviceIdType.MESH,
                )

            def y_fwd(k):
                s = base + k * rows
                return pltpu.make_async_remote_copy(
                    src_ref=out_ref.at[0, pl.ds(s, rows), :],
                    dst_ref=out_ref.at[0, pl.ds(s, rows), :],
                    send_sem=y_send.at[k],
                    recv_sem=y_recv.at[k],
                    device_id=(my_x, 1 - my_y),
                    device_id_type=pl.DeviceIdType.MESH,
                )

            def y_in(k):
                s = other + k * rows
                return pltpu.make_async_remote_copy(
                    src_ref=out_ref.at[0, pl.ds(s, rows), :],
                    dst_ref=out_ref.at[0, pl.ds(s, rows), :],
                    send_sem=y_send.at[k],
                    recv_sem=y_recv.at[k],
                    device_id=(my_x, 1 - my_y),
                    device_id_type=pl.DeviceIdType.MESH,
                )

            for k in range(C):
                x_desc(k).start()
            for k in range(C):
                x_desc(k).wait_recv()
                y_fwd(k).start()
            for k in range(C):
                y_in(k).wait_recv()
            for k in range(C):
                x_desc(k).wait_send()
                y_fwd(k).wait_send()

        @pl.when(dst_x == my_x)
        def _():
            cp = pltpu.make_async_copy(x_ref, out_ref, copy_sem)
            cp.start()
            cp.wait()

    return pl.pallas_call(
        body,
        out_shape=jax.ShapeDtypeStruct(x.shape, x.dtype),
        in_specs=[
            pl.BlockSpec(memory_space=pl.ANY),
            pl.BlockSpec(memory_space=pltpu.MemorySpace.SMEM),
        ],
        out_specs=pl.BlockSpec(memory_space=pl.ANY),
        scratch_shapes=[
            pltpu.SemaphoreType.DMA((C,)),
            pltpu.SemaphoreType.DMA((C,)),
            pltpu.SemaphoreType.DMA((C,)),
            pltpu.SemaphoreType.DMA((C,)),
            pltpu.SemaphoreType.DMA,
        ],
        compiler_params=pltpu.CompilerParams(collective_id=0),
    )(x, pi)


# device time: 215804 ns/iter; 1.8114x vs baseline; 1.0074x over previous
import jax
import jax.numpy as jnp
from jax import lax
from jax.experimental import pallas as pl
from jax.experimental.pallas import tpu as pltpu

C = 64


def kernel(x, pi):
    _, M, N = x.shape
    half = M // 2
    rows = half // C

    def body(x_ref, pi_ref, out_ref, x_send, x_recv, y_send, y_recv, copy_sem):
        my_x = lax.axis_index("x")
        my_y = lax.axis_index("y")
        dst_x = pi_ref[my_x]

        barrier = pltpu.get_barrier_semaphore()
        pl.semaphore_signal(barrier, inc=1, device_id=(1 - my_x, my_y),
                            device_id_type=pl.DeviceIdType.MESH)
        pl.semaphore_signal(barrier, inc=1, device_id=(my_x, 1 - my_y),
                            device_id_type=pl.DeviceIdType.MESH)
        pl.semaphore_wait(barrier, 2)

        @pl.when(dst_x != my_x)
        def _():
            base = my_y * half
            other = (1 - my_y) * half

            def x_desc(k):
                s = base + k * rows
                return pltpu.make_async_remote_copy(
                    src_ref=x_ref.at[0, pl.ds(s, rows), :],
                    dst_ref=out_ref.at[0, pl.ds(s, rows), :],
                    send_sem=x_send.at[k],
                    recv_sem=x_recv.at[k],
                    device_id=(dst_x, my_y),
                    device_id_type=pl.DeviceIdType.MESH,
                )

            def y_fwd(k):
                s = base + k * rows
                return pltpu.make_async_remote_copy(
                    src_ref=out_ref.at[0, pl.ds(s, rows), :],
                    dst_ref=out_ref.at[0, pl.ds(s, rows), :],
                    send_sem=y_send.at[k],
                    recv_sem=y_recv.at[k],
                    device_id=(my_x, 1 - my_y),
                    device_id_type=pl.DeviceIdType.MESH,
                )

            def y_in(k):
                s = other + k * rows
                return pltpu.make_async_remote_copy(
                    src_ref=out_ref.at[0, pl.ds(s, rows), :],
                    dst_ref=out_ref.at[0, pl.ds(s, rows), :],
                    send_sem=y_send.at[k],
                    recv_sem=y_recv.at[k],
                    device_id=(my_x, 1 - my_y),
                    device_id_type=pl.DeviceIdType.MESH,
                )

            for k in range(C):
                x_desc(k).start()
            for k in range(C):
                x_desc(k).wait_recv()
                y_fwd(k).start()
            for k in range(C):
                y_in(k).wait_recv()
            for k in range(C):
                x_desc(k).wait_send()
                y_fwd(k).wait_send()

        @pl.when(dst_x == my_x)
        def _():
            cp = pltpu.make_async_copy(x_ref, out_ref, copy_sem)
            cp.start()
            cp.wait()

    return pl.pallas_call(
        body,
        out_shape=jax.ShapeDtypeStruct(x.shape, x.dtype),
        in_specs=[
            pl.BlockSpec(memory_space=pl.ANY),
            pl.BlockSpec(memory_space=pltpu.MemorySpace.SMEM),
        ],
        out_specs=pl.BlockSpec(memory_space=pl.ANY),
        scratch_shapes=[
            pltpu.SemaphoreType.DMA((C,)),
            pltpu.SemaphoreType.DMA((C,)),
            pltpu.SemaphoreType.DMA((C,)),
            pltpu.SemaphoreType.DMA((C,)),
            pltpu.SemaphoreType.DMA,
        ],
        compiler_params=pltpu.CompilerParams(collective_id=0),
    )(x, pi)
